# Initial kernel scaffold; baseline (speedup 1.0000x reference)
import jax
import jax.numpy as jnp
from jax.experimental import pallas as pl


def kernel(x, edge_index, W1, b1, W2, b2, W3, b3):
    raise NotImplementedError("write your pallas kernel here")


# speedup vs baseline: 2.7658x; 2.7658x over previous
import functools

import jax
import jax.numpy as jnp
from jax import lax
from jax.experimental import pallas as pl
from jax.experimental.pallas import tpu as pltpu
from jax.experimental.pallas import tpu_sc as plsc

N = 10000
E = 320000
DIN = 128
DHID = 256
NC, NS = 2, 16
NW = NC * NS
EB = 128
CH = 2560
EPAD = CH * EB
NPAD = 10240
ROWS_ZERO = NPAD // NS


def _make_segsum(table_rows, cpw):
    mesh = plsc.VectorSubcoreMesh(core_axis_name="c", subcore_axis_name="s")

    @functools.partial(
        pl.kernel,
        out_type=jax.ShapeDtypeStruct((NC, NPAD, DIN), jnp.float32),
        mesh=mesh,
        scratch_types=[
            pltpu.VMEM_SHARED((NPAD, DIN), jnp.float32),
            pltpu.VMEM((EB,), jnp.int32),
            pltpu.VMEM((EB,), jnp.int32),
            pltpu.VMEM((EB, DIN), jnp.float32),
            pltpu.SemaphoreType.DMA,
        ],
    )
    def segsum(table, srcs, dsts, zeros, out, acc, src_v, dst_v, rows_v, sem):
        c = lax.axis_index("c")
        s = lax.axis_index("s")
        pltpu.sync_copy(zeros.at[pl.ds(s * ROWS_ZERO, ROWS_ZERO)],
                        acc.at[pl.ds(s * ROWS_ZERO, ROWS_ZERO)])
        plsc.subcore_barrier()

        def body(i, carry):
            pltpu.sync_copy(srcs.at[c, s, i], src_v)
            pltpu.sync_copy(dsts.at[c, s, i], dst_v)
            pltpu.async_copy(table.at[src_v], rows_v, sem).wait()
            pltpu.sync_copy(rows_v, acc.at[dst_v], add=True)
            return carry

        lax.fori_loop(0, cpw, body, 0)
        plsc.subcore_barrier()
        pltpu.sync_copy(acc.at[pl.ds(s * ROWS_ZERO, ROWS_ZERO)],
                        out.at[c, pl.ds(s * ROWS_ZERO, ROWS_ZERO)])

    return segsum


_segsum_split_edges = _make_segsum(N, CH // NW)
_segsum_split_cols = _make_segsum(NC * N, CH // NS)


def _mlp_body(xb, a1b, w1, b1, w2, b2, out):
    z = xb[...] + a1b[0] + a1b[1]
    t = jnp.dot(z, w1[...], preferred_element_type=jnp.float32) + b1[...]
    t = jnp.maximum(t, 0.0)
    h = jnp.dot(t, w2[...], preferred_element_type=jnp.float32) + b2[...]
    h = jnp.maximum(h, 0.0)
    out[0] = h[:, :DIN]
    out[1] = h[:, DIN:]


def _final_body(hsb, a2b, w3, b3, out):
    u0 = hsb[0] + a2b[0]
    u1 = hsb[1] + a2b[1]
    out[...] = (jnp.dot(u0, w3[0], preferred_element_type=jnp.float32)
                + jnp.dot(u1, w3[1], preferred_element_type=jnp.float32)
                + b3[...])


_MLP_R = 1000


def _mlp(x, parts, w1, b1, w2, b2):
    grid = N // _MLP_R
    return pl.pallas_call(
        _mlp_body,
        grid=(grid,),
        in_specs=[
            pl.BlockSpec((_MLP_R, DIN), lambda i: (i, 0)),
            pl.BlockSpec((NC, _MLP_R, DIN), lambda i: (0, i, 0)),
            pl.BlockSpec((DIN, DHID), lambda i: (0, 0)),
            pl.BlockSpec((1, DHID), lambda i: (0, 0)),
            pl.BlockSpec((DHID, DHID), lambda i: (0, 0)),
            pl.BlockSpec((1, DHID), lambda i: (0, 0)),
        ],
        out_specs=pl.BlockSpec((NC, _MLP_R, DIN), lambda i: (0, i, 0)),
        out_shape=jax.ShapeDtypeStruct((NC, N, DIN), jnp.float32),
    )(x, parts, w1, b1, w2, b2)


def _final(hs, a2, w3, b3):
    grid = N // _MLP_R
    return pl.pallas_call(
        _final_body,
        grid=(grid,),
        in_specs=[
            pl.BlockSpec((NC, _MLP_R, DIN), lambda i: (0, i, 0)),
            pl.BlockSpec((NC, _MLP_R, DIN), lambda i: (0, i, 0)),
            pl.BlockSpec((NC, DIN, DHID), lambda i: (0, 0, 0)),
            pl.BlockSpec((1, DHID), lambda i: (0, 0)),
        ],
        out_specs=pl.BlockSpec((_MLP_R, DHID), lambda i: (i, 0)),
        out_shape=jax.ShapeDtypeStruct((N, DHID), jnp.float32),
    )(hs, a2, w3, b3)


def kernel(x, edge_index, W1, b1, W2, b2, W3, b3):
    src = edge_index[0].astype(jnp.int32)
    dst = edge_index[1].astype(jnp.int32)
    npad_e = EPAD - E
    src_p = jnp.concatenate([src, jnp.zeros((npad_e,), jnp.int32)])
    dst_p = jnp.concatenate([dst, jnp.full((npad_e,), N, jnp.int32)])
    zeros = jnp.zeros((NPAD, DIN), jnp.float32)

    s4 = src_p.reshape(NC, NS, CH // NW, EB)
    d4 = dst_p.reshape(NC, NS, CH // NW, EB)
    parts = _segsum_split_edges(x, s4, d4, zeros)

    hs = _mlp(x, parts, W1, b1.reshape(1, DHID), W2, b2.reshape(1, DHID))

    s3 = src_p.reshape(NS, CH // NS, EB)
    d3 = dst_p.reshape(NS, CH // NS, EB)
    s4b = jnp.stack([s3, s3 + N])
    d4b = jnp.stack([d3, d3])
    a2 = _segsum_split_cols(hs.reshape(NC * N, DIN), s4b, d4b, zeros)

    return _final(hs, a2, W3.reshape(NC, DIN, DHID), b3.reshape(1, DHID))


# speedup vs baseline: 3.0135x; 1.0895x over previous
import functools

import jax
import jax.numpy as jnp
from jax import lax
from jax.experimental import pallas as pl
from jax.experimental.pallas import tpu as pltpu
from jax.experimental.pallas import tpu_sc as plsc

N = 10000
E = 320000
DIN = 128
DHID = 256
NC, NS = 2, 16
NW = NC * NS
EB = 64
NB = 4
EPAD = 327680
NPAD = 10240
ROWS_ZERO = NPAD // NS


def _make_segsum(table_rows, rounds):
    mesh = plsc.VectorSubcoreMesh(core_axis_name="c", subcore_axis_name="s")

    @functools.partial(
        pl.kernel,
        out_type=jax.ShapeDtypeStruct((NC, NPAD, DIN), jnp.float32),
        mesh=mesh,
        scratch_types=[
            pltpu.VMEM_SHARED((NPAD, DIN), jnp.float32),
            pltpu.VMEM((2, NB, EB), jnp.int32),
            pltpu.VMEM((NB, EB, DIN), jnp.float32),
            pltpu.SemaphoreType.DMA,
            pltpu.SemaphoreType.DMA,
        ],
    )
    def segsum(table, sd, zeros, out, acc, sdbuf, rows, gsem, ssem):
        c = lax.axis_index("c")
        s = lax.axis_index("s")
        pltpu.sync_copy(zeros.at[pl.ds(s * ROWS_ZERO, ROWS_ZERO)],
                        acc.at[pl.ds(s * ROWS_ZERO, ROWS_ZERO)])
        plsc.subcore_barrier()

        def body(j, carry):
            pltpu.sync_copy(sd.at[c, s, j], sdbuf)
            gd = [pltpu.async_copy(table.at[sdbuf.at[0, b]],
                                   rows.at[b], gsem) for b in range(NB)]
            for d in gd:
                d.wait()
            sc = [pltpu.async_copy(rows.at[b], acc.at[sdbuf.at[1, b]],
                                   ssem, add=True) for b in range(NB)]
            for d in sc:
                d.wait()
            return carry

        lax.fori_loop(0, rounds, body, 0)
        plsc.subcore_barrier()
        pltpu.sync_copy(acc.at[pl.ds(s * ROWS_ZERO, ROWS_ZERO)],
                        out.at[c, pl.ds(s * ROWS_ZERO, ROWS_ZERO)])

    return segsum


R_AGG1 = EPAD // (NW * NB * EB)
R_AGG2 = EPAD // (NS * NB * EB)

_segsum_split_edges = _make_segsum(N, R_AGG1)
_segsum_split_cols = _make_segsum(NC * N, R_AGG2)


def _mlp_body(xb, a1b, w1, b1, w2, b2, out):
    z = xb[...] + a1b[0] + a1b[1]
    t = jnp.dot(z, w1[...], preferred_element_type=jnp.float32) + b1[...]
    t = jnp.maximum(t, 0.0)
    h = jnp.dot(t, w2[...], preferred_element_type=jnp.float32) + b2[...]
    h = jnp.maximum(h, 0.0)
    out[0] = h[:, :DIN]
    out[1] = h[:, DIN:]


def _final_body(hsb, a2b, w3, b3, out):
    u0 = hsb[0] + a2b[0]
    u1 = hsb[1] + a2b[1]
    out[...] = (jnp.dot(u0, w3[0], preferred_element_type=jnp.float32)
                + jnp.dot(u1, w3[1], preferred_element_type=jnp.float32)
                + b3[...])


_MLP_R = 1000


def _mlp(x, parts, w1, b1, w2, b2):
    grid = N // _MLP_R
    return pl.pallas_call(
        _mlp_body,
        grid=(grid,),
        in_specs=[
            pl.BlockSpec((_MLP_R, DIN), lambda i: (i, 0)),
            pl.BlockSpec((NC, _MLP_R, DIN), lambda i: (0, i, 0)),
            pl.BlockSpec((DIN, DHID), lambda i: (0, 0)),
            pl.BlockSpec((1, DHID), lambda i: (0, 0)),
            pl.BlockSpec((DHID, DHID), lambda i: (0, 0)),
            pl.BlockSpec((1, DHID), lambda i: (0, 0)),
        ],
        out_specs=pl.BlockSpec((NC, _MLP_R, DIN), lambda i: (0, i, 0)),
        out_shape=jax.ShapeDtypeStruct((NC, N, DIN), jnp.float32),
    )(x, parts, w1, b1, w2, b2)


def _final(hs, a2, w3, b3):
    grid = N // _MLP_R
    return pl.pallas_call(
        _final_body,
        grid=(grid,),
        in_specs=[
            pl.BlockSpec((NC, _MLP_R, DIN), lambda i: (0, i, 0)),
            pl.BlockSpec((NC, _MLP_R, DIN), lambda i: (0, i, 0)),
            pl.BlockSpec((NC, DIN, DHID), lambda i: (0, 0, 0)),
            pl.BlockSpec((1, DHID), lambda i: (0, 0)),
        ],
        out_specs=pl.BlockSpec((_MLP_R, DHID), lambda i: (i, 0)),
        out_shape=jax.ShapeDtypeStruct((N, DHID), jnp.float32),
    )(hs, a2, w3, b3)


def kernel(x, edge_index, W1, b1, W2, b2, W3, b3):
    src = edge_index[0].astype(jnp.int32)
    dst = edge_index[1].astype(jnp.int32)
    npad_e = EPAD - E
    src_p = jnp.concatenate([src, jnp.zeros((npad_e,), jnp.int32)])
    dst_p = jnp.concatenate([dst, jnp.full((npad_e,), N, jnp.int32)])
    zeros = jnp.zeros((NPAD, DIN), jnp.float32)

    s1 = src_p.reshape(NC, NS, R_AGG1, NB, EB)
    d1 = dst_p.reshape(NC, NS, R_AGG1, NB, EB)
    sd1 = jnp.stack([s1, d1], axis=3)
    parts = _segsum_split_edges(x, sd1, zeros)

    hs = _mlp(x, parts, W1, b1.reshape(1, DHID), W2, b2.reshape(1, DHID))

    s3 = src_p.reshape(NS, R_AGG2, NB, EB)
    d3 = dst_p.reshape(NS, R_AGG2, NB, EB)
    s2 = jnp.stack([s3, s3 + N])
    d2 = jnp.stack([d3, d3])
    sd2 = jnp.stack([s2, d2], axis=3)
    a2 = _segsum_split_cols(hs.reshape(NC * N, DIN), sd2, zeros)

    return _final(hs, a2, W3.reshape(NC, DIN, DHID), b3.reshape(1, DHID))


# speedup vs baseline: 3.0764x; 1.0209x over previous
import functools

import jax
import jax.numpy as jnp
from jax import lax
from jax.experimental import pallas as pl
from jax.experimental.pallas import tpu as pltpu
from jax.experimental.pallas import tpu_sc as plsc

N = 10000
E = 320000
DIN = 128
DHID = 256
NC, NS = 2, 16
NW = NC * NS
EB = 64
KP = 2
NB = 4
EPAD = 327680
NPAD = 10240
ROWS_ZERO = NPAD // NS


def _make_segsum(table_rows, rounds):
    mesh = plsc.VectorSubcoreMesh(core_axis_name="c", subcore_axis_name="s")

    @functools.partial(
        pl.kernel,
        out_type=jax.ShapeDtypeStruct((NC, NPAD, DIN), jnp.float32),
        mesh=mesh,
        scratch_types=[
            pltpu.VMEM_SHARED((NPAD, DIN), jnp.float32),
            pltpu.VMEM((2, 2, KP, EB), jnp.int32),
            pltpu.VMEM((2, KP, EB, DIN), jnp.float32),
            pltpu.SemaphoreType.DMA,
            pltpu.SemaphoreType.DMA,
            pltpu.SemaphoreType.DMA,
        ],
    )
    def segsum(table, sd, zeros, out, acc, sdbuf, rows, gsem, ssem0, ssem1):
        c = lax.axis_index("c")
        s = lax.axis_index("s")
        pltpu.sync_copy(zeros.at[pl.ds(s * ROWS_ZERO, ROWS_ZERO)],
                        acc.at[pl.ds(s * ROWS_ZERO, ROWS_ZERO)])
        plsc.subcore_barrier()

        def drain(p, ssem):
            for b in range(KP):
                pltpu.make_async_copy(rows.at[p, b],
                                      acc.at[sdbuf.at[p, 1, b]], ssem).wait()

        def stage(t, p, ssem):
            @pl.when(t > 0)
            def _():
                drain(p, ssem)
            pltpu.sync_copy(sd.at[c, s, t, p], sdbuf.at[p])
            gd = [pltpu.async_copy(table.at[sdbuf.at[p, 0, b]],
                                   rows.at[p, b], gsem) for b in range(KP)]
            for d in gd:
                d.wait()
            for b in range(KP):
                pltpu.async_copy(rows.at[p, b], acc.at[sdbuf.at[p, 1, b]],
                                 ssem, add=True)

        def body(t, carry):
            stage(t, 0, ssem0)
            stage(t, 1, ssem1)
            return carry

        lax.fori_loop(0, rounds, body, 0)
        drain(0, ssem0)
        drain(1, ssem1)
        plsc.subcore_barrier()
        pltpu.sync_copy(acc.at[pl.ds(s * ROWS_ZERO, ROWS_ZERO)],
                        out.at[c, pl.ds(s * ROWS_ZERO, ROWS_ZERO)])

    return segsum


R_AGG1 = EPAD // (NW * NB * EB)
R_AGG2 = EPAD // (NS * NB * EB)

_segsum_split_edges = _make_segsum(N, R_AGG1)
_segsum_split_cols = _make_segsum(NC * N, R_AGG2)


def _mlp_body(xb, a1b, w1, b1, w2, b2, out):
    z = xb[...] + a1b[0] + a1b[1]
    t = jnp.dot(z, w1[...], preferred_element_type=jnp.float32) + b1[...]
    t = jnp.maximum(t, 0.0)
    h = jnp.dot(t, w2[...], preferred_element_type=jnp.float32) + b2[...]
    h = jnp.maximum(h, 0.0)
    out[0] = h[:, :DIN]
    out[1] = h[:, DIN:]


def _final_body(hsb, a2b, w3, b3, out):
    u0 = hsb[0] + a2b[0]
    u1 = hsb[1] + a2b[1]
    out[...] = (jnp.dot(u0, w3[0], preferred_element_type=jnp.float32)
                + jnp.dot(u1, w3[1], preferred_element_type=jnp.float32)
                + b3[...])


_MLP_R = 1000


def _mlp(x, parts, w1, b1, w2, b2):
    grid = N // _MLP_R
    return pl.pallas_call(
        _mlp_body,
        grid=(grid,),
        in_specs=[
            pl.BlockSpec((_MLP_R, DIN), lambda i: (i, 0)),
            pl.BlockSpec((NC, _MLP_R, DIN), lambda i: (0, i, 0)),
            pl.BlockSpec((DIN, DHID), lambda i: (0, 0)),
            pl.BlockSpec((1, DHID), lambda i: (0, 0)),
            pl.BlockSpec((DHID, DHID), lambda i: (0, 0)),
            pl.BlockSpec((1, DHID), lambda i: (0, 0)),
        ],
        out_specs=pl.BlockSpec((NC, _MLP_R, DIN), lambda i: (0, i, 0)),
        out_shape=jax.ShapeDtypeStruct((NC, N, DIN), jnp.float32),
    )(x, parts, w1, b1, w2, b2)


def _final(hs, a2, w3, b3):
    grid = N // _MLP_R
    return pl.pallas_call(
        _final_body,
        grid=(grid,),
        in_specs=[
            pl.BlockSpec((NC, _MLP_R, DIN), lambda i: (0, i, 0)),
            pl.BlockSpec((NC, _MLP_R, DIN), lambda i: (0, i, 0)),
            pl.BlockSpec((NC, DIN, DHID), lambda i: (0, 0, 0)),
            pl.BlockSpec((1, DHID), lambda i: (0, 0)),
        ],
        out_specs=pl.BlockSpec((_MLP_R, DHID), lambda i: (i, 0)),
        out_shape=jax.ShapeDtypeStruct((N, DHID), jnp.float32),
    )(hs, a2, w3, b3)


def kernel(x, edge_index, W1, b1, W2, b2, W3, b3):
    src = edge_index[0].astype(jnp.int32)
    dst = edge_index[1].astype(jnp.int32)
    npad_e = EPAD - E
    src_p = jnp.concatenate([src, jnp.zeros((npad_e,), jnp.int32)])
    dst_p = jnp.concatenate([dst, jnp.full((npad_e,), N, jnp.int32)])
    zeros = jnp.zeros((NPAD, DIN), jnp.float32)

    s1 = src_p.reshape(NC, NS, R_AGG1, 2, KP, EB)
    d1 = dst_p.reshape(NC, NS, R_AGG1, 2, KP, EB)
    sd1 = jnp.stack([s1, d1], axis=4)
    parts = _segsum_split_edges(x, sd1, zeros)

    hs = _mlp(x, parts, W1, b1.reshape(1, DHID), W2, b2.reshape(1, DHID))

    s3 = src_p.reshape(NS, R_AGG2, 2, KP, EB)
    d3 = dst_p.reshape(NS, R_AGG2, 2, KP, EB)
    s2 = jnp.stack([s3, s3 + N])
    d2 = jnp.stack([d3, d3])
    sd2 = jnp.stack([s2, d2], axis=4)
    a2 = _segsum_split_cols(hs.reshape(NC * N, DIN), sd2, zeros)

    return _final(hs, a2, W3.reshape(NC, DIN, DHID), b3.reshape(1, DHID))


# speedup vs baseline: 4.3832x; 1.4248x over previous
import functools

import jax
import jax.numpy as jnp
from jax import lax
from jax.experimental import pallas as pl
from jax.experimental.pallas import tpu as pltpu
from jax.experimental.pallas import tpu_sc as plsc

N = 10000
NH = 5000
E = 320000
DIN = 128
DHID = 256
NC, NS = 2, 16
EB = 24
SB = 6
EPAD = 322560
SLABS = EPAD // (NS * SB * EB)
ACC_R = 5120
ACC_ZR = ACC_R // NS
ST_R = 624


def _make_segsum(n_passes, sd_by_core):
    mesh = plsc.VectorSubcoreMesh(core_axis_name="c", subcore_axis_name="s")

    @functools.partial(
        pl.kernel,
        out_type=jax.ShapeDtypeStruct((NC, n_passes, ACC_R, DIN),
                                      jnp.float32),
        mesh=mesh,
        scratch_types=[
            pltpu.VMEM_SHARED((N, DIN), jnp.float32),
            pltpu.VMEM_SHARED((ACC_R, DIN), jnp.float32),
            pltpu.VMEM((2, SB, 2, EB), jnp.int32),
            pltpu.VMEM((2, EB, DIN), jnp.float32),
            pltpu.SemaphoreType.DMA,
            pltpu.SemaphoreType.DMA,
            pltpu.SemaphoreType.DMA,
            pltpu.SemaphoreType.DMA,
            pltpu.SemaphoreType.DMA,
        ],
    )
    def segsum(tables, sd, zeros, out, spt, acc, sdb, rows,
               gsem, ssem0, ssem1, isem0, isem1):
        c = lax.axis_index("c")
        s = lax.axis_index("s")
        ssems = (ssem0, ssem1)
        isems = (isem0, isem1)

        pltpu.sync_copy(tables.at[c, pl.ds(s * ST_R, ST_R)],
                        spt.at[pl.ds(s * ST_R, ST_R)])

        @pl.when(s == 0)
        def _():
            pltpu.sync_copy(tables.at[c, pl.ds(NS * ST_R, N - NS * ST_R)],
                            spt.at[pl.ds(NS * ST_R, N - NS * ST_R)])

        for q in range(n_passes):
            sd_q = sd.at[c] if sd_by_core else sd.at[q]
            pltpu.sync_copy(zeros.at[pl.ds(s * ACC_ZR, ACC_ZR)],
                            acc.at[pl.ds(s * ACC_ZR, ACC_ZR)])
            plsc.subcore_barrier()

            pltpu.async_copy(sd_q.at[s, 0], sdb.at[0], isem0)

            def slab(t, p, drain_scatter):
                pltpu.make_async_copy(sd_q.at[s, t], sdb.at[p],
                                      isems[p]).wait()
                for b in range(SB):
                    rb = b % 2
                    if drain_scatter or b >= 2:
                        pltpu.make_async_copy(
                            rows.at[rb], acc.at[sdb.at[p, b, 1]],
                            ssems[rb]).wait()
                    if b == 2:
                        tn = jnp.minimum(t + 1, SLABS - 1)
                        pltpu.async_copy(sd_q.at[s, tn], sdb.at[1 - p],
                                         isems[1 - p])
                    pltpu.async_copy(spt.at[sdb.at[p, b, 0]],
                                     rows.at[rb], gsem).wait()
                    pltpu.async_copy(rows.at[rb], acc.at[sdb.at[p, b, 1]],
                                     ssems[rb], add=True)

            slab(jnp.int32(0), 0, False)

            def body(j, carry):
                t = 1 + 2 * j
                slab(t, 1, True)
                slab(t + 1, 0, True)
                return carry

            lax.fori_loop(0, (SLABS - 2) // 2, body, 0)
            slab(jnp.int32(SLABS - 1), 1, True)
            for rb in range(2):
                pltpu.make_async_copy(rows.at[rb], acc.at[pl.ds(0, EB)],
                                      ssems[rb]).wait()
            pltpu.make_async_copy(sd_q.at[s, SLABS - 1],
                                  sdb.at[0], isems[0]).wait()
            plsc.subcore_barrier()
            pltpu.sync_copy(acc.at[pl.ds(s * ACC_ZR, ACC_ZR)],
                            out.at[c, q, pl.ds(s * ACC_ZR, ACC_ZR)])

    return segsum


_segsum_agg1 = _make_segsum(1, True)
_segsum_agg2 = _make_segsum(2, False)


def _mlp_body(xb, a1b, w1, b1, w2, b2, out):
    z = xb[...] + a1b[0, 0]
    t = jnp.dot(z, w1[...], preferred_element_type=jnp.float32) + b1[...]
    t = jnp.maximum(t, 0.0)
    h = jnp.dot(t, w2[...], preferred_element_type=jnp.float32) + b2[...]
    h = jnp.maximum(h, 0.0)
    out[0] = h[:, :DIN]
    out[1] = h[:, DIN:]


def _final_body(hb, a2b, w3, b3, out):
    u0 = hb[0] + a2b[0, 0]
    u1 = hb[1] + a2b[1, 0]
    out[...] = (jnp.dot(u0, w3[0], preferred_element_type=jnp.float32)
                + jnp.dot(u1, w3[1], preferred_element_type=jnp.float32)
                + b3[...])


_MLP_R = 1000


def _mlp(x, a1, w1, b1, w2, b2):
    grid = N // _MLP_R
    nb = NH // _MLP_R
    return pl.pallas_call(
        _mlp_body,
        grid=(grid,),
        in_specs=[
            pl.BlockSpec((_MLP_R, DIN), lambda i: (i, 0)),
            pl.BlockSpec((1, 1, _MLP_R, DIN), lambda i: (i // nb, 0, i % nb, 0)),
            pl.BlockSpec((DIN, DHID), lambda i: (0, 0)),
            pl.BlockSpec((1, DHID), lambda i: (0, 0)),
            pl.BlockSpec((DHID, DHID), lambda i: (0, 0)),
            pl.BlockSpec((1, DHID), lambda i: (0, 0)),
        ],
        out_specs=pl.BlockSpec((NC, _MLP_R, DIN), lambda i: (0, i, 0)),
        out_shape=jax.ShapeDtypeStruct((NC, N, DIN), jnp.float32),
    )(x, a1, w1, b1, w2, b2)


def _final(h, a2, w3, b3):
    grid = N // _MLP_R
    nb = NH // _MLP_R
    return pl.pallas_call(
        _final_body,
        grid=(grid,),
        in_specs=[
            pl.BlockSpec((NC, _MLP_R, DIN), lambda i: (0, i, 0)),
            pl.BlockSpec((NC, 1, _MLP_R, DIN), lambda i: (0, i // nb, i % nb, 0)),
            pl.BlockSpec((NC, DIN, DHID), lambda i: (0, 0, 0)),
            pl.BlockSpec((1, DHID), lambda i: (0, 0)),
        ],
        out_specs=pl.BlockSpec((_MLP_R, DHID), lambda i: (i, 0)),
        out_shape=jax.ShapeDtypeStruct((N, DHID), jnp.float32),
    )(h, a2, w3, b3)


def _block_idx(src_p, dst_half):
    s_r = src_p.reshape(NS, SLABS, SB, EB)
    d_r = dst_half.reshape(NS, SLABS, SB, EB)
    return jnp.stack([s_r, d_r], axis=3)


def kernel(x, edge_index, W1, b1, W2, b2, W3, b3):
    src = edge_index[0].astype(jnp.int32)
    dst = edge_index[1].astype(jnp.int32)
    npad_e = EPAD - E
    src_p = jnp.concatenate([src, jnp.zeros((npad_e,), jnp.int32)])
    dst_p = jnp.concatenate([dst, jnp.full((npad_e,), N, jnp.int32)])
    discard = NH + (jnp.arange(EPAD, dtype=jnp.int32) % (ACC_R - NH))

    def remap(h):
        lo = h * NH
        inr = (dst_p >= lo) & (dst_p < lo + NH)
        return jnp.where(inr, dst_p - lo, discard)

    sd1 = jnp.stack([_block_idx(src_p, remap(0)),
                     _block_idx(src_p, remap(1))])
    zeros = jnp.zeros((ACC_R, DIN), jnp.float32)

    a1 = _segsum_agg1(jnp.broadcast_to(x, (NC, N, DIN)), sd1, zeros)
    h = _mlp(x, a1, W1, b1.reshape(1, DHID), W2, b2.reshape(1, DHID))
    a2 = _segsum_agg2(h, sd1, zeros)
    return _final(h, a2, W3.reshape(NC, DIN, DHID), b3.reshape(1, DHID))
